# Initial kernel scaffold; baseline (speedup 1.0000x reference)
import jax
import jax.numpy as jnp
from jax.experimental import pallas as pl


def kernel(input, weight_q, weight_r):
    raise NotImplementedError("write your pallas kernel here")


# speedup vs baseline: 2.8709x; 2.8709x over previous
import functools

import jax
import jax.numpy as jnp
from jax import lax
from jax.experimental import pallas as pl
from jax.experimental.pallas import tpu as pltpu
from jax.experimental.pallas import tpu_sc as plsc

_NUM_COLLISIONS = 1024
_SHIFT = 10
_MASK = _NUM_COLLISIONS - 1
_EMBED_DIM = 64
_BATCH = 16384
_NC = 2
_NS = 16
_NW = _NC * _NS
_BPW = _BATCH // _NW
_LANES = 16


@functools.cache
def _build():
    @functools.partial(
        pl.kernel,
        out_type=jax.ShapeDtypeStruct((_BATCH, _EMBED_DIM), jnp.float32),
        mesh=plsc.VectorSubcoreMesh(core_axis_name="c", subcore_axis_name="s"),
        scratch_types=[
            pltpu.VMEM((_BPW,), jnp.int32),
            pltpu.VMEM((_BPW,), jnp.int32),
            pltpu.VMEM((_BPW,), jnp.int32),
            pltpu.VMEM((_BPW, _EMBED_DIM), jnp.float32),
            pltpu.VMEM((_BPW, _EMBED_DIM), jnp.float32),
            pltpu.SemaphoreType.DMA,
            pltpu.SemaphoreType.DMA,
        ],
        compiler_params=pltpu.CompilerParams(use_tc_tiling_on_sc=False),
    )
    def _qr_embed(idx_hbm, wq_hbm, wr_hbm, out_hbm,
                  idx_v, q_v, r_v, rows_q, rows_r, sem_q, sem_r):
        wid = lax.axis_index("s") * _NC + lax.axis_index("c")
        base = wid * _BPW
        pltpu.sync_copy(idx_hbm.at[pl.ds(base, _BPW)], idx_v)

        def split_body(i, carry):
            sl = pl.ds(i * _LANES, _LANES)
            v = idx_v[sl]
            q_v[sl] = lax.shift_right_logical(v, _SHIFT)
            r_v[sl] = lax.bitwise_and(v, _MASK)
            return carry

        lax.fori_loop(0, _BPW // _LANES, split_body, 0)

        cp_q = pltpu.async_copy(wq_hbm.at[q_v], rows_q, sem_q)
        cp_r = pltpu.async_copy(wr_hbm.at[r_v], rows_r, sem_r)
        cp_q.wait()
        cp_r.wait()

        def mul_body(row, carry):
            for j in range(_EMBED_DIM // _LANES):
                sl = pl.ds(j * _LANES, _LANES)
                rows_q[row, sl] = rows_q[row, sl] * rows_r[row, sl]
            return carry

        lax.fori_loop(0, _BPW, mul_body, 0)

        pltpu.sync_copy(rows_q, out_hbm.at[pl.ds(base, _BPW)])

    return _qr_embed


def kernel(input, weight_q, weight_r):
    return _build()(input, weight_q, weight_r)
